# Initial kernel scaffold; baseline (speedup 1.0000x reference)
import jax
import jax.numpy as jnp
from jax.experimental import pallas as pl


def kernel(x, edge_index, batch, ratios, W1, b1, g1, be1, W2, b2, g2, be2, mtW1, mtb1, mtW2, mtb2, rW1, rb1, rW2, rb2, rW3, rb3, rW4, rb4):
    raise NotImplementedError("write your pallas kernel here")


# speedup vs baseline: 12.1190x; 12.1190x over previous
import functools
import numpy as np
import jax
import jax.numpy as jnp
from jax import lax
from jax.experimental import pallas as pl
from jax.experimental.pallas import tpu as pltpu
from jax.experimental.pallas import tpu_sc as plsc

NN = 10000
NP = 10240
EE = 320000
HH = 128
MM = 512
NC = 2
NS = 16
NW = NC * NS
EC = EE // NW
K = 80
NCHUNK = EC // K
RPT = NP // NS
ISQ = float(1.0 / np.sqrt(1.0 + 1e-5))

_mesh = plsc.VectorSubcoreMesh(core_axis_name="c", subcore_axis_name="s")
_SC_PARAMS = pltpu.CompilerParams(needs_layout_passes=False)


def _i16(v):
    return v + jnp.zeros((16,), jnp.int32)


_DROWS = NP // 128


@functools.partial(
    pl.kernel,
    out_type=jax.ShapeDtypeStruct((NC, _DROWS, 128), jnp.float32),
    mesh=_mesh,
    compiler_params=_SC_PARAMS,
    scratch_types=[
        pltpu.VMEM((K,), jnp.int32),
        pltpu.VMEM((_DROWS,), jnp.int32),
        pltpu.VMEM((_DROWS, 128), jnp.float32),
        pltpu.VMEM_SHARED((_DROWS, 128), jnp.float32),
    ],
)
def _deg(dst_hbm, iden_hbm, z80_hbm, out_hbm, didx, iden, dacc, acc):
    c = lax.axis_index("c")
    s = lax.axis_index("s")
    wid = c * NS + s

    @pl.when(s < _DROWS // 8)
    def _():
        pltpu.sync_copy(z80_hbm.at[pl.ds(s * 8, 8)], acc.at[pl.ds(s * 8, 8)])

    pltpu.sync_copy(z80_hbm, dacc)
    pltpu.sync_copy(iden_hbm, iden)
    plsc.subcore_barrier()

    iota16 = lax.iota(jnp.int32, 16)
    ones16 = jnp.ones((16,), jnp.float32)

    def body(i, _):
        base = pl.multiple_of(wid * EC + i * K, 8)
        pltpu.sync_copy(dst_hbm.at[pl.ds(base, K)], didx)
        for g in range(K // 16):
            dvec = didx[pl.ds(g * 16, 16)]
            hi = lax.shift_right_logical(dvec, 7)
            lo = jnp.bitwise_and(dvec, 127)
            for j in range(16):
                plsc.addupdate_scatter(dacc, [hi, lo], ones16,
                                       mask=iota16 == j)
        return 0

    lax.fori_loop(0, NCHUNK, body, 0)
    pltpu.sync_copy(dacc, acc.at[iden], add=True)
    plsc.subcore_barrier()

    @pl.when(s < _DROWS // 8)
    def _():
        pltpu.sync_copy(acc.at[pl.ds(s * 8, 8)],
                        out_hbm.at[c, pl.ds(s * 8, 8)])


@functools.partial(
    pl.kernel,
    out_type=jax.ShapeDtypeStruct((NC, NP, HH), jnp.float32),
    mesh=_mesh,
    compiler_params=_SC_PARAMS,
    scratch_types=[
        pltpu.VMEM((K,), jnp.int32),
        pltpu.VMEM((K,), jnp.int32),
        pltpu.VMEM((K, HH), jnp.float32),
        pltpu.VMEM_SHARED((NP, HH), jnp.float32),
        pltpu.SemaphoreType.DMA,
    ],
)
def _prop(hs_hbm, src_hbm, dst_hbm, znh_hbm, out_hbm, sidx, didx, rows, acc, sem):
    c = lax.axis_index("c")
    s = lax.axis_index("s")
    wid = c * NS + s
    pltpu.sync_copy(znh_hbm.at[pl.ds(s * RPT, RPT)], acc.at[pl.ds(s * RPT, RPT)])
    plsc.subcore_barrier()

    def body(i, _):
        base = pl.multiple_of(wid * EC + i * K, 8)
        pltpu.sync_copy(src_hbm.at[pl.ds(base, K)], sidx)
        pltpu.sync_copy(dst_hbm.at[pl.ds(base, K)], didx)
        pltpu.async_copy(hs_hbm.at[sidx], rows, sem).wait()
        pltpu.sync_copy(rows, acc.at[didx], add=True)
        return 0

    lax.fori_loop(0, NCHUNK, body, 0)
    plsc.subcore_barrier()
    pltpu.sync_copy(acc.at[pl.ds(s * RPT, RPT)],
                    out_hbm.at[c, pl.ds(s * RPT, RPT)])


_POOL_ACTIVE = 25
_POOL_ROWS = 400
_MPT = MM // NS


_CROWS = MM // 128


@functools.partial(
    pl.kernel,
    out_type=[
        jax.ShapeDtypeStruct((NC, MM, HH), jnp.float32),
        jax.ShapeDtypeStruct((NC, _CROWS, 128), jnp.float32),
        jax.ShapeDtypeStruct((NW, MM, HH), jnp.float32),
    ],
    mesh=_mesh,
    compiler_params=_SC_PARAMS,
    scratch_types=[
        pltpu.VMEM((K,), jnp.int32),
        pltpu.VMEM((K, 1), jnp.float32),
        pltpu.VMEM((K, HH), jnp.float32),
        pltpu.VMEM((_CROWS,), jnp.int32),
        pltpu.VMEM((_CROWS, 128), jnp.float32),
        pltpu.VMEM((MM, HH), jnp.float32),
        pltpu.VMEM_SHARED((MM, HH), jnp.float32),
        pltpu.VMEM_SHARED((_CROWS, 128), jnp.float32),
    ],
)
def _pool(z_hbm, batch_hbm, batchf_hbm, iden_hbm, zmh_hbm,
          osum_hbm, ocnt_hbm, omax_hbm,
          bidx, bfv, rows, iden, cacc, maxacc, sacc, cacc_sp):
    c = lax.axis_index("c")
    s = lax.axis_index("s")
    wid = c * NS + s
    pltpu.sync_copy(zmh_hbm.at[pl.ds(s * _MPT, _MPT)],
                    sacc.at[pl.ds(s * _MPT, _MPT)])
    pltpu.sync_copy(zmh_hbm.at[pl.ds(0, _CROWS)], cacc)
    pltpu.sync_copy(zmh_hbm, maxacc)
    pltpu.sync_copy(iden_hbm, iden)

    @pl.when(s == 0)
    def _():
        pltpu.sync_copy(zmh_hbm.at[pl.ds(0, _CROWS)], cacc_sp)

    plsc.subcore_barrier()

    iota16 = lax.iota(jnp.int32, 16)
    ones16 = jnp.ones((16,), jnp.float32)
    mask0 = iota16 == 0

    @pl.when(wid < _POOL_ACTIVE)
    def _():
        def chunk(i, _):
            base = pl.multiple_of(wid * _POOL_ROWS + i * K, 8)
            pltpu.sync_copy(batch_hbm.at[pl.ds(base, K)], bidx)
            pltpu.sync_copy(batchf_hbm.at[pl.ds(base, K)], bfv)
            pltpu.sync_copy(z_hbm.at[pl.ds(base, K)], rows)
            pltpu.sync_copy(rows, sacc.at[bidx], add=True)

            def rowfn(r, _2):
                bm = plsc.load_gather(bfv, [_i16(r), _i16(0)]).astype(jnp.int32)
                plsc.addupdate_scatter(
                    cacc, [lax.shift_right_logical(bm, 7),
                           jnp.bitwise_and(bm, 127)], ones16, mask=mask0)
                for cc in range(HH // 16):
                    colv = iota16 + cc * 16
                    v = plsc.load_gather(rows, [_i16(r), colv])
                    cur = plsc.load_gather(maxacc, [bm, colv])
                    plsc.store_scatter(maxacc, [bm, colv], jnp.maximum(cur, v))
                return 0

            lax.fori_loop(0, K, rowfn, 0)
            return 0

        lax.fori_loop(0, _POOL_ROWS // K, chunk, 0)

    pltpu.sync_copy(cacc, cacc_sp.at[iden], add=True)
    plsc.subcore_barrier()
    pltpu.sync_copy(sacc.at[pl.ds(s * _MPT, _MPT)],
                    osum_hbm.at[c, pl.ds(s * _MPT, _MPT)])

    @pl.when(s == 0)
    def _():
        pltpu.sync_copy(cacc_sp, ocnt_hbm.at[c])

    pltpu.sync_copy(maxacc, omax_hbm.at[wid])


_BLK = 2048
_GRID = NP // _BLK


def _t1_body(x_ref, w_ref, d0_ref, d1_ref, hs_ref, dinv_ref):
    deg = 1.0 + d0_ref[...] + d1_ref[...]
    dinv = lax.rsqrt(deg)
    h = jnp.dot(x_ref[...], w_ref[...], preferred_element_type=jnp.float32)
    hs_ref[...] = h * dinv
    dinv_ref[...] = dinv


def _t1(x, w1, d0, d1):
    return pl.pallas_call(
        _t1_body,
        grid=(_GRID,),
        in_specs=[
            pl.BlockSpec((_BLK, HH), lambda i: (i, 0)),
            pl.BlockSpec((HH, HH), lambda i: (0, 0)),
            pl.BlockSpec((_BLK, 1), lambda i: (i, 0)),
            pl.BlockSpec((_BLK, 1), lambda i: (i, 0)),
        ],
        out_specs=[
            pl.BlockSpec((_BLK, HH), lambda i: (i, 0)),
            pl.BlockSpec((_BLK, 1), lambda i: (i, 0)),
        ],
        out_shape=[
            jax.ShapeDtypeStruct((NP, HH), jnp.float32),
            jax.ShapeDtypeStruct((NP, 1), jnp.float32),
        ],
    )(x, w1, d0, d1)


def _t2_body(p0_ref, p1_ref, hs_ref, dinv_ref, w_ref, b_ref, g_ref, be_ref,
             out_ref):
    dinv = dinv_ref[...]
    conv = dinv * (p0_ref[...] + p1_ref[...] + hs_ref[...]) + b_ref[...]
    z = jax.nn.relu(g_ref[...] * (conv * ISQ) + be_ref[...])
    out_ref[...] = jnp.dot(z, w_ref[...],
                           preferred_element_type=jnp.float32) * dinv


def _t2(p0, p1, hs, dinv, w2, b1, g1, be1):
    return pl.pallas_call(
        _t2_body,
        grid=(_GRID,),
        in_specs=[
            pl.BlockSpec((_BLK, HH), lambda i: (i, 0)),
            pl.BlockSpec((_BLK, HH), lambda i: (i, 0)),
            pl.BlockSpec((_BLK, HH), lambda i: (i, 0)),
            pl.BlockSpec((_BLK, 1), lambda i: (i, 0)),
            pl.BlockSpec((HH, HH), lambda i: (0, 0)),
            pl.BlockSpec((1, HH), lambda i: (0, 0)),
            pl.BlockSpec((1, HH), lambda i: (0, 0)),
            pl.BlockSpec((1, HH), lambda i: (0, 0)),
        ],
        out_specs=pl.BlockSpec((_BLK, HH), lambda i: (i, 0)),
        out_shape=jax.ShapeDtypeStruct((NP, HH), jnp.float32),
    )(p0, p1, hs, dinv, w2, b1, g1, be1)


def _t3_body(q0_ref, q1_ref, hs_ref, dinv_ref, b_ref, g_ref, be_ref, out_ref):
    conv = dinv_ref[...] * (q0_ref[...] + q1_ref[...] + hs_ref[...]) + b_ref[...]
    out_ref[...] = jax.nn.relu(g_ref[...] * (conv * ISQ) + be_ref[...])


def _t3(q0, q1, hs, dinv, b2, g2, be2):
    return pl.pallas_call(
        _t3_body,
        grid=(_GRID,),
        in_specs=[
            pl.BlockSpec((_BLK, HH), lambda i: (i, 0)),
            pl.BlockSpec((_BLK, HH), lambda i: (i, 0)),
            pl.BlockSpec((_BLK, HH), lambda i: (i, 0)),
            pl.BlockSpec((_BLK, 1), lambda i: (i, 0)),
            pl.BlockSpec((1, HH), lambda i: (0, 0)),
            pl.BlockSpec((1, HH), lambda i: (0, 0)),
            pl.BlockSpec((1, HH), lambda i: (0, 0)),
        ],
        out_specs=pl.BlockSpec((_BLK, HH), lambda i: (i, 0)),
        out_shape=jax.ShapeDtypeStruct((NP, HH), jnp.float32),
    )(q0, q1, hs, dinv, b2, g2, be2)


def _t4_body(sp_ref, c0_ref, c1_ref, mp_ref, ratios_ref,
             mtw1_ref, mtb1_ref, mtw2_ref, mtb2_ref,
             rw1_ref, rb1_ref, rw2_ref, rb2_ref,
             rw3_ref, rb3_ref, rw4_ref, rb4_ref, out_ref):
    sums = sp_ref[0] + sp_ref[1]
    cnt = c0_ref[...] + c1_ref[...]
    mx = jnp.max(mp_ref[...], axis=0)
    mean = sums / jnp.maximum(cnt, 1.0)
    mol = jnp.concatenate([mean, mx, sums], axis=1)
    rows = lax.broadcasted_iota(jnp.int32, (HH, MM), 0)
    cols = lax.broadcasted_iota(jnp.int32, (HH, MM), 1)
    rm = jnp.zeros((HH, MM), jnp.float32)
    for j in range(4):
        rm = rm + jnp.where(cols == 4 * rows + j, ratios_ref[:, j:j + 1], 0.0)
    mix = jnp.dot(rm, mol, preferred_element_type=jnp.float32)
    f = jax.nn.relu(jnp.dot(mix, mtw1_ref[...],
                            preferred_element_type=jnp.float32) + mtb1_ref[...])
    f = jax.nn.relu(jnp.dot(f, mtw2_ref[...],
                            preferred_element_type=jnp.float32) + mtb2_ref[...])
    r = jax.nn.relu(jnp.dot(f, rw1_ref[...],
                            preferred_element_type=jnp.float32) + rb1_ref[...])
    r = jax.nn.relu(jnp.dot(r, rw2_ref[...],
                            preferred_element_type=jnp.float32) + rb2_ref[...])
    r = jax.nn.relu(jnp.dot(r, rw3_ref[...],
                            preferred_element_type=jnp.float32) + rb3_ref[...])
    out_ref[...] = jnp.dot(r, rw4_ref[...],
                           preferred_element_type=jnp.float32) + rb4_ref[...]


def _t4(sp, c0, c1, mp, ratios, mtw1, mtb1, mtw2, mtb2,
        rw1, rb1, rw2, rb2, rw3, rb3, rw4, rb4):
    return pl.pallas_call(
        _t4_body,
        out_shape=jax.ShapeDtypeStruct((128, 1), jnp.float32),
    )(sp, c0, c1, mp, ratios, mtw1, mtb1, mtw2, mtb2,
      rw1, rb1, rw2, rb2, rw3, rb3, rw4, rb4)


def kernel(x, edge_index, batch, ratios, W1, b1, g1, be1, W2, b2, g2, be2,
           mtW1, mtb1, mtW2, mtb2, rW1, rb1, rW2, rb2, rW3, rb3, rW4, rb4):
    src = edge_index[0]
    dst = edge_index[1]
    x = jnp.pad(x, ((0, NP - NN), (0, 0)))
    iden80 = jnp.arange(_DROWS, dtype=jnp.int32)
    iden4 = jnp.arange(_CROWS, dtype=jnp.int32)
    z80 = jnp.zeros((_DROWS, 128), jnp.float32)
    znh = jnp.zeros((NP, HH), jnp.float32)
    zmh = jnp.zeros((MM, HH), jnp.float32)

    degp = _deg(dst, iden80, z80)
    hs1, dinv = _t1(x, W1, degp[0].reshape(NP, 1), degp[1].reshape(NP, 1))
    p = _prop(hs1, src, dst, znh)
    hs2 = _t2(p[0], p[1], hs1, dinv, W2,
              b1.reshape(1, HH), g1.reshape(1, HH), be1.reshape(1, HH))
    q = _prop(hs2, src, dst, znh)
    z2 = _t3(q[0], q[1], hs2, dinv,
             b2.reshape(1, HH), g2.reshape(1, HH), be2.reshape(1, HH))
    batchf = batch.astype(jnp.float32).reshape(NN, 1)
    sp, cp, mp = _pool(z2, batch, batchf, iden4, zmh)
    return _t4(sp, cp[0].reshape(MM, 1), cp[1].reshape(MM, 1), mp, ratios,
               mtW1, mtb1.reshape(1, -1), mtW2, mtb2.reshape(1, -1),
               rW1, rb1.reshape(1, -1), rW2, rb2.reshape(1, -1),
               rW3, rb3.reshape(1, -1), rW4, rb4.reshape(1, -1))


# speedup vs baseline: 20.6960x; 1.7077x over previous
import functools
import numpy as np
import jax
import jax.numpy as jnp
from jax import lax
from jax.experimental import pallas as pl
from jax.experimental.pallas import tpu as pltpu
from jax.experimental.pallas import tpu_sc as plsc

NN = 10000
NP = 10240
EE = 320000
HH = 128
MM = 512
NC = 2
NS = 16
NW = NC * NS
EC = EE // NW
K = 80
NCHUNK = EC // K
RPT = NP // NS
ISQ = float(1.0 / np.sqrt(1.0 + 1e-5))

_mesh = plsc.VectorSubcoreMesh(core_axis_name="c", subcore_axis_name="s")
_SC_PARAMS = pltpu.CompilerParams(needs_layout_passes=False)


def _i16(v):
    return v + jnp.zeros((16,), jnp.int32)


_DROWS = NP // 128


@functools.partial(
    pl.kernel,
    out_type=jax.ShapeDtypeStruct((NC, _DROWS, 128), jnp.float32),
    mesh=_mesh,
    compiler_params=_SC_PARAMS,
    scratch_types=[
        pltpu.VMEM((EC,), jnp.int32),
        pltpu.VMEM((_DROWS,), jnp.int32),
        pltpu.VMEM((_DROWS, 128), jnp.float32),
        pltpu.VMEM_SHARED((_DROWS, 128), jnp.float32),
    ],
)
def _deg(dst_hbm, iden_hbm, z80_hbm, out_hbm, didx_all, iden, dacc, acc):
    c = lax.axis_index("c")
    s = lax.axis_index("s")
    wid = c * NS + s

    @pl.when(s < _DROWS // 8)
    def _():
        pltpu.sync_copy(z80_hbm.at[pl.ds(s * 8, 8)], acc.at[pl.ds(s * 8, 8)])

    pltpu.sync_copy(z80_hbm, dacc)
    pltpu.sync_copy(iden_hbm, iden)
    pltpu.sync_copy(dst_hbm.at[pl.ds(pl.multiple_of(wid * EC, 8), EC)],
                    didx_all)
    plsc.subcore_barrier()

    iota16 = lax.iota(jnp.int32, 16)
    ones16 = jnp.ones((16,), jnp.float32)

    def body(i, _):
        voff = pl.multiple_of(i * K, 8)
        for g in range(K // 16):
            dvec = didx_all[pl.ds(voff + g * 16, 16)]
            hi = lax.shift_right_logical(dvec, 7)
            lo = jnp.bitwise_and(dvec, 127)
            for j in range(16):
                plsc.addupdate_scatter(dacc, [hi, lo], ones16,
                                       mask=iota16 == j)
        return 0

    lax.fori_loop(0, NCHUNK, body, 0)
    pltpu.sync_copy(dacc, acc.at[iden], add=True)
    plsc.subcore_barrier()

    @pl.when(s < _DROWS // 8)
    def _():
        pltpu.sync_copy(acc.at[pl.ds(s * 8, 8)],
                        out_hbm.at[c, pl.ds(s * 8, 8)])


NBUF = 5
KP = 40
NCHUNKP = EC // KP


@functools.partial(
    pl.kernel,
    out_type=jax.ShapeDtypeStruct((NC, NP, HH), jnp.float32),
    mesh=_mesh,
    compiler_params=_SC_PARAMS,
    scratch_types=(
        [pltpu.VMEM((EC,), jnp.int32)]
        + [pltpu.VMEM((KP,), jnp.int32) for _ in range(NBUF)]
        + [pltpu.VMEM((KP, HH), jnp.float32) for _ in range(NBUF)]
        + [pltpu.SemaphoreType.DMA] * (3 * NBUF)
        + [pltpu.VMEM_SHARED((NP, HH), jnp.float32)]
    ),
)
def _prop(hs_hbm, src_hbm, dst_hbm, znh_hbm, out_hbm,
          sidx_all, di0, di1, di2, di3, di4, ro0, ro1, ro2, ro3, ro4,
          ds0, ds1, ds2, ds3, ds4, gs0, gs1, gs2, gs3, gs4,
          ss0, ss1, ss2, ss3, ss4, acc):
    didx = [di0, di1, di2, di3, di4]
    rows = [ro0, ro1, ro2, ro3, ro4]
    dsem = [ds0, ds1, ds2, ds3, ds4]
    gsem = [gs0, gs1, gs2, gs3, gs4]
    ssem = [ss0, ss1, ss2, ss3, ss4]
    c = lax.axis_index("c")
    s = lax.axis_index("s")
    wid = c * NS + s
    ebase = pl.multiple_of(wid * EC, 8)
    pltpu.sync_copy(znh_hbm.at[pl.ds(s * RPT, RPT)], acc.at[pl.ds(s * RPT, RPT)])
    pltpu.sync_copy(src_hbm.at[pl.ds(ebase, EC)], sidx_all)
    plsc.subcore_barrier()

    def issue(ci, b):
        base = pl.multiple_of(ebase + ci * KP, 8)
        voff = pl.multiple_of(ci * KP, 8)
        pltpu.async_copy(dst_hbm.at[pl.ds(base, KP)], didx[b], dsem[b])
        pltpu.async_copy(hs_hbm.at[sidx_all.at[pl.ds(voff, KP)]],
                         rows[b], gsem[b])

    issue(0, 0)

    def outer(g, _):
        for b in range(NBUF):
            ci = g * NBUF + b
            nb = (b + 1) % NBUF

            @pl.when(ci + 1 < NCHUNKP)
            def _():
                @pl.when(ci + 1 >= NBUF)
                def _():
                    pltpu.make_async_copy(rows[nb], acc.at[didx[nb]],
                                          ssem[nb]).wait()

                issue(ci + 1, nb)

            base = pl.multiple_of(ebase + ci * KP, 8)
            voff = pl.multiple_of(ci * KP, 8)
            pltpu.make_async_copy(dst_hbm.at[pl.ds(base, KP)], didx[b],
                                  dsem[b]).wait()
            pltpu.make_async_copy(hs_hbm.at[sidx_all.at[pl.ds(voff, KP)]],
                                  rows[b], gsem[b]).wait()
            pltpu.async_copy(rows[b], acc.at[didx[b]], ssem[b], add=True)
        return 0

    lax.fori_loop(0, NCHUNKP // NBUF, outer, 0)
    for b in range(NBUF):
        pltpu.make_async_copy(rows[b], acc.at[didx[b]], ssem[b]).wait()
    plsc.subcore_barrier()
    pltpu.sync_copy(acc.at[pl.ds(s * RPT, RPT)],
                    out_hbm.at[c, pl.ds(s * RPT, RPT)])


_POOL_ACTIVE = 25
_POOL_ROWS = 400
_MPT = MM // NS


_CROWS = MM // 128


@functools.partial(
    pl.kernel,
    out_type=[
        jax.ShapeDtypeStruct((NC, MM, HH), jnp.float32),
        jax.ShapeDtypeStruct((NC, _CROWS, 128), jnp.float32),
        jax.ShapeDtypeStruct((NW, MM, HH), jnp.float32),
    ],
    mesh=_mesh,
    compiler_params=_SC_PARAMS,
    scratch_types=[
        pltpu.VMEM((K,), jnp.int32),
        pltpu.VMEM((K, 1), jnp.float32),
        pltpu.VMEM((K, HH), jnp.float32),
        pltpu.VMEM((_CROWS,), jnp.int32),
        pltpu.VMEM((_CROWS, 128), jnp.float32),
        pltpu.VMEM((MM, HH), jnp.float32),
        pltpu.VMEM_SHARED((MM, HH), jnp.float32),
        pltpu.VMEM_SHARED((_CROWS, 128), jnp.float32),
    ],
)
def _pool(z_hbm, batch_hbm, batchf_hbm, iden_hbm, zmh_hbm,
          osum_hbm, ocnt_hbm, omax_hbm,
          bidx, bfv, rows, iden, cacc, maxacc, sacc, cacc_sp):
    c = lax.axis_index("c")
    s = lax.axis_index("s")
    wid = c * NS + s
    pltpu.sync_copy(zmh_hbm.at[pl.ds(s * _MPT, _MPT)],
                    sacc.at[pl.ds(s * _MPT, _MPT)])
    pltpu.sync_copy(zmh_hbm.at[pl.ds(0, _CROWS)], cacc)
    pltpu.sync_copy(zmh_hbm, maxacc)
    pltpu.sync_copy(iden_hbm, iden)

    @pl.when(s == 0)
    def _():
        pltpu.sync_copy(zmh_hbm.at[pl.ds(0, _CROWS)], cacc_sp)

    plsc.subcore_barrier()

    iota16 = lax.iota(jnp.int32, 16)
    ones16 = jnp.ones((16,), jnp.float32)
    mask0 = iota16 == 0

    @pl.when(wid < _POOL_ACTIVE)
    def _():
        def chunk(i, _):
            base = pl.multiple_of(wid * _POOL_ROWS + i * K, 8)
            pltpu.sync_copy(batch_hbm.at[pl.ds(base, K)], bidx)
            pltpu.sync_copy(batchf_hbm.at[pl.ds(base, K)], bfv)
            pltpu.sync_copy(z_hbm.at[pl.ds(base, K)], rows)
            pltpu.sync_copy(rows, sacc.at[bidx], add=True)

            def rowfn(r, _2):
                bm = plsc.load_gather(bfv, [_i16(r), _i16(0)]).astype(jnp.int32)
                plsc.addupdate_scatter(
                    cacc, [lax.shift_right_logical(bm, 7),
                           jnp.bitwise_and(bm, 127)], ones16, mask=mask0)
                for cc in range(HH // 16):
                    colv = iota16 + cc * 16
                    v = plsc.load_gather(rows, [_i16(r), colv])
                    cur = plsc.load_gather(maxacc, [bm, colv])
                    plsc.store_scatter(maxacc, [bm, colv], jnp.maximum(cur, v))
                return 0

            lax.fori_loop(0, K, rowfn, 0)
            return 0

        lax.fori_loop(0, _POOL_ROWS // K, chunk, 0)

    pltpu.sync_copy(cacc, cacc_sp.at[iden], add=True)
    plsc.subcore_barrier()
    pltpu.sync_copy(sacc.at[pl.ds(s * _MPT, _MPT)],
                    osum_hbm.at[c, pl.ds(s * _MPT, _MPT)])

    @pl.when(s == 0)
    def _():
        pltpu.sync_copy(cacc_sp, ocnt_hbm.at[c])

    pltpu.sync_copy(maxacc, omax_hbm.at[wid])


_BLK = 2048
_GRID = NP // _BLK


def _t1_body(x_ref, w_ref, d0_ref, d1_ref, hs_ref, dinv_ref):
    deg = 1.0 + d0_ref[...] + d1_ref[...]
    dinv = lax.rsqrt(deg)
    h = jnp.dot(x_ref[...], w_ref[...], preferred_element_type=jnp.float32)
    hs_ref[...] = h * dinv
    dinv_ref[...] = dinv


def _t1(x, w1, d0, d1):
    return pl.pallas_call(
        _t1_body,
        grid=(_GRID,),
        in_specs=[
            pl.BlockSpec((_BLK, HH), lambda i: (i, 0)),
            pl.BlockSpec((HH, HH), lambda i: (0, 0)),
            pl.BlockSpec((_BLK, 1), lambda i: (i, 0)),
            pl.BlockSpec((_BLK, 1), lambda i: (i, 0)),
        ],
        out_specs=[
            pl.BlockSpec((_BLK, HH), lambda i: (i, 0)),
            pl.BlockSpec((_BLK, 1), lambda i: (i, 0)),
        ],
        out_shape=[
            jax.ShapeDtypeStruct((NP, HH), jnp.float32),
            jax.ShapeDtypeStruct((NP, 1), jnp.float32),
        ],
    )(x, w1, d0, d1)


def _t2_body(p0_ref, p1_ref, hs_ref, dinv_ref, w_ref, b_ref, g_ref, be_ref,
             out_ref):
    dinv = dinv_ref[...]
    conv = dinv * (p0_ref[...] + p1_ref[...] + hs_ref[...]) + b_ref[...]
    z = jax.nn.relu(g_ref[...] * (conv * ISQ) + be_ref[...])
    out_ref[...] = jnp.dot(z, w_ref[...],
                           preferred_element_type=jnp.float32) * dinv


def _t2(p0, p1, hs, dinv, w2, b1, g1, be1):
    return pl.pallas_call(
        _t2_body,
        grid=(_GRID,),
        in_specs=[
            pl.BlockSpec((_BLK, HH), lambda i: (i, 0)),
            pl.BlockSpec((_BLK, HH), lambda i: (i, 0)),
            pl.BlockSpec((_BLK, HH), lambda i: (i, 0)),
            pl.BlockSpec((_BLK, 1), lambda i: (i, 0)),
            pl.BlockSpec((HH, HH), lambda i: (0, 0)),
            pl.BlockSpec((1, HH), lambda i: (0, 0)),
            pl.BlockSpec((1, HH), lambda i: (0, 0)),
            pl.BlockSpec((1, HH), lambda i: (0, 0)),
        ],
        out_specs=pl.BlockSpec((_BLK, HH), lambda i: (i, 0)),
        out_shape=jax.ShapeDtypeStruct((NP, HH), jnp.float32),
    )(p0, p1, hs, dinv, w2, b1, g1, be1)


def _t3_body(q0_ref, q1_ref, hs_ref, dinv_ref, b_ref, g_ref, be_ref, out_ref):
    conv = dinv_ref[...] * (q0_ref[...] + q1_ref[...] + hs_ref[...]) + b_ref[...]
    out_ref[...] = jax.nn.relu(g_ref[...] * (conv * ISQ) + be_ref[...])


def _t3(q0, q1, hs, dinv, b2, g2, be2):
    return pl.pallas_call(
        _t3_body,
        grid=(_GRID,),
        in_specs=[
            pl.BlockSpec((_BLK, HH), lambda i: (i, 0)),
            pl.BlockSpec((_BLK, HH), lambda i: (i, 0)),
            pl.BlockSpec((_BLK, HH), lambda i: (i, 0)),
            pl.BlockSpec((_BLK, 1), lambda i: (i, 0)),
            pl.BlockSpec((1, HH), lambda i: (0, 0)),
            pl.BlockSpec((1, HH), lambda i: (0, 0)),
            pl.BlockSpec((1, HH), lambda i: (0, 0)),
        ],
        out_specs=pl.BlockSpec((_BLK, HH), lambda i: (i, 0)),
        out_shape=jax.ShapeDtypeStruct((NP, HH), jnp.float32),
    )(q0, q1, hs, dinv, b2, g2, be2)


def _t4_body(sp0_ref, sp1_ref, sp2_ref, sp3_ref,
             cp0_ref, cp1_ref, cp2_ref, cp3_ref,
             mp0_ref, mp1_ref, mp2_ref, mp3_ref, ratios_ref,
             mtw1_ref, mtb1_ref, mtw2_ref, mtb2_ref,
             rw1_ref, rb1_ref, rw2_ref, rb2_ref,
             rw3_ref, rb3_ref, rw4_ref, rb4_ref, out_ref):
    mix = jnp.zeros((HH, 3 * HH), jnp.float32)
    for j, (sp, cp, mp) in enumerate([
            (sp0_ref, cp0_ref, mp0_ref), (sp1_ref, cp1_ref, mp1_ref),
            (sp2_ref, cp2_ref, mp2_ref), (sp3_ref, cp3_ref, mp3_ref)]):
        sums_j = sp[0] + sp[1]
        cnt_j = cp[0] + cp[1]
        mx_j = jnp.max(mp[...], axis=0)
        mean_j = sums_j / jnp.maximum(cnt_j, 1.0)
        mol_j = jnp.concatenate([mean_j, mx_j, sums_j], axis=1)
        mix = mix + ratios_ref[:, j:j + 1] * mol_j
    f = jax.nn.relu(jnp.dot(mix, mtw1_ref[...],
                            preferred_element_type=jnp.float32) + mtb1_ref[...])
    f = jax.nn.relu(jnp.dot(f, mtw2_ref[...],
                            preferred_element_type=jnp.float32) + mtb2_ref[...])
    r = jax.nn.relu(jnp.dot(f, rw1_ref[...],
                            preferred_element_type=jnp.float32) + rb1_ref[...])
    r = jax.nn.relu(jnp.dot(r, rw2_ref[...],
                            preferred_element_type=jnp.float32) + rb2_ref[...])
    r = jax.nn.relu(jnp.dot(r, rw3_ref[...],
                            preferred_element_type=jnp.float32) + rb3_ref[...])
    out_ref[...] = jnp.dot(r, rw4_ref[...],
                           preferred_element_type=jnp.float32) + rb4_ref[...]


def _t4(sps, cps, mps, ratios, mtw1, mtb1, mtw2, mtb2,
        rw1, rb1, rw2, rb2, rw3, rb3, rw4, rb4):
    return pl.pallas_call(
        _t4_body,
        out_shape=jax.ShapeDtypeStruct((128, 1), jnp.float32),
    )(*sps, *cps, *mps, ratios, mtw1, mtb1, mtw2, mtb2,
      rw1, rb1, rw2, rb2, rw3, rb3, rw4, rb4)


def kernel(x, edge_index, batch, ratios, W1, b1, g1, be1, W2, b2, g2, be2,
           mtW1, mtb1, mtW2, mtb2, rW1, rb1, rW2, rb2, rW3, rb3, rW4, rb4):
    src = edge_index[0]
    dst = edge_index[1]
    x = jnp.pad(x, ((0, NP - NN), (0, 0)))
    iden80 = jnp.arange(_DROWS, dtype=jnp.int32)
    iden4 = jnp.arange(_CROWS, dtype=jnp.int32)
    z80 = jnp.zeros((_DROWS, 128), jnp.float32)
    znh = jnp.zeros((NP, HH), jnp.float32)
    zmh = jnp.zeros((MM, HH), jnp.float32)

    degp = _deg(dst, iden80, z80)
    hs1, dinv = _t1(x, W1, degp[0].reshape(NP, 1), degp[1].reshape(NP, 1))
    p = _prop(hs1, src, dst, znh)
    hs2 = _t2(p[0], p[1], hs1, dinv, W2,
              b1.reshape(1, HH), g1.reshape(1, HH), be1.reshape(1, HH))
    q = _prop(hs2, src, dst, znh)
    z2 = _t3(q[0], q[1], hs2, dinv,
             b2.reshape(1, HH), g2.reshape(1, HH), be2.reshape(1, HH))
    batchf = batch.astype(jnp.float32).reshape(NN, 1)
    sp, cp, mp = _pool(z2, batch, batchf, iden4, zmh)
    cnt2 = cp.reshape(NC, MM, 1)
    sps = [sp[:, j::4, :] for j in range(4)]
    cps = [cnt2[:, j::4, :] for j in range(4)]
    mps = [mp[:, j::4, :] for j in range(4)]
    return _t4(sps, cps, mps, ratios,
               mtW1, mtb1.reshape(1, -1), mtW2, mtb2.reshape(1, -1),
               rW1, rb1.reshape(1, -1), rW2, rb2.reshape(1, -1),
               rW3, rb3.reshape(1, -1), rW4, rb4.reshape(1, -1))
